# Initial kernel scaffold; baseline (speedup 1.0000x reference)
import jax
import jax.numpy as jnp
from jax.experimental import pallas as pl


def kernel(x, edge_index, W_enc, b_enc, W_dec, b_dec):
    raise NotImplementedError("write your pallas kernel here")


# speedup vs baseline: 4.3562x; 4.3562x over previous
import functools

import jax
import jax.numpy as jnp
from jax import lax
from jax.experimental import pallas as pl
from jax.experimental.pallas import tpu as pltpu
from jax.experimental.pallas import tpu_sc as plsc

NC = 2
NS = 16
NW = NC * NS
CHUNK = 128


def _sc_scatter_kernel(n_pad, d, cpw, x_shape):
    mesh = plsc.VectorSubcoreMesh(core_axis_name="c", subcore_axis_name="s")
    rows_per_tile = n_pad // NS

    @functools.partial(
        pl.kernel,
        out_type=jax.ShapeDtypeStruct((NC, n_pad, d), jnp.float32),
        mesh=mesh,
        scratch_types=[
            pltpu.VMEM_SHARED((n_pad, d), jnp.float32),
            pltpu.VMEM((CHUNK,), jnp.int32),
            pltpu.VMEM((CHUNK,), jnp.int32),
            pltpu.VMEM((CHUNK, d), jnp.float32),
            pltpu.SemaphoreType.DMA,
        ],
    )
    def sc_kernel(x_hbm, srcp_hbm, dstp_hbm, zero_hbm, out_hbm,
                  h_sh, sidx, didx, rows, gsem):
        c = lax.axis_index("c")
        s = lax.axis_index("s")
        wid = s * NC + c
        r0 = s * rows_per_tile
        pltpu.sync_copy(zero_hbm.at[pl.ds(r0, rows_per_tile)],
                        h_sh.at[pl.ds(r0, rows_per_tile)])
        plsc.subcore_barrier()

        def body(j, carry):
            pltpu.sync_copy(srcp_hbm.at[wid, j], sidx)
            pltpu.sync_copy(dstp_hbm.at[wid, j], didx)
            pltpu.async_copy(x_hbm.at[sidx], rows, gsem).wait()
            pltpu.sync_copy(rows, h_sh.at[didx], add=True)
            return carry

        lax.fori_loop(0, cpw, body, 0)
        plsc.subcore_barrier()
        pltpu.sync_copy(h_sh.at[pl.ds(r0, rows_per_tile)],
                        out_hbm.at[c, pl.ds(r0, rows_per_tile)])

    return sc_kernel


def _tc_dense_kernel(p_ref, we_ref, be_ref, wd_ref, bd_ref, o_ref):
    h = p_ref[0] + p_ref[1]
    lat = jnp.dot(h, we_ref[...], preferred_element_type=jnp.float32)
    lat = jnp.maximum(lat + be_ref[...], 0.0)
    rec = jnp.dot(lat, wd_ref[...], preferred_element_type=jnp.float32)
    rec = rec + bd_ref[...]
    e = jnp.exp(rec)
    o_ref[...] = e / jnp.sum(e, axis=-1, keepdims=True)


def kernel(x, edge_index, W_enc, b_enc, W_dec, b_dec):
    n, d = x.shape
    e = edge_index.shape[1]
    lat_dim = W_enc.shape[1]

    n_pad = ((n + 8 * NS) + (128 * NS - 1)) // (128 * NS) * (128 * NS)
    cpw = -(-e // (NW * CHUNK))
    e_pad = NW * cpw * CHUNK

    src = edge_index[0]
    dst = edge_index[1]
    srcp = jnp.concatenate(
        [src, jnp.zeros((e_pad - e,), jnp.int32)]).reshape(NW, cpw, CHUNK)
    dstp = jnp.concatenate(
        [dst, jnp.full((e_pad - e,), n, jnp.int32)]).reshape(NW, cpw, CHUNK)
    zero = jnp.zeros((n_pad, d), jnp.float32)

    partials = _sc_scatter_kernel(n_pad, d, cpw, x.shape)(x, srcp, dstp, zero)

    grid = 4
    br = n_pad // grid
    prob = pl.pallas_call(
        _tc_dense_kernel,
        grid=(grid,),
        in_specs=[
            pl.BlockSpec((NC, br, d), lambda i: (0, i, 0)),
            pl.BlockSpec((d, lat_dim), lambda i: (0, 0)),
            pl.BlockSpec((1, lat_dim), lambda i: (0, 0)),
            pl.BlockSpec((lat_dim, d), lambda i: (0, 0)),
            pl.BlockSpec((1, d), lambda i: (0, 0)),
        ],
        out_specs=pl.BlockSpec((br, d), lambda i: (i, 0)),
        out_shape=jax.ShapeDtypeStruct((n_pad, d), jnp.float32),
    )(partials, W_enc, b_enc.reshape(1, lat_dim), W_dec, b_dec.reshape(1, d))

    return prob[:n]
